# baseline (speedup 1.0000x reference)
import jax
import jax.numpy as jnp
from jax import lax
from jax.experimental import pallas as pl

NUM_CLASSES = 80
N = 134400
BLOCK = 8960


def _criterion_block(pred_cls_ref, r_ref, p_ref, g_ref, a_ref, lab_ref,
                     cls_ref, clsm_ref, reg_ref, box_ref, npos_ref):
    i = pl.program_id(0)

    @pl.when(i == 0)
    def _init():
        cls_ref[...] = jnp.zeros_like(cls_ref)
        clsm_ref[...] = jnp.zeros_like(clsm_ref)
        reg_ref[...] = jnp.zeros_like(reg_ref)
        box_ref[...] = jnp.zeros_like(box_ref)
        npos_ref[...] = jnp.zeros_like(npos_ref)

    lrow = lab_ref[...]
    pos_row = (lrow >= 0.0) & (lrow < float(NUM_CLASSES))
    pos_f = pos_row.astype(jnp.float32)

    labels = lrow.reshape(BLOCK, 1).astype(jnp.int32)
    posb = (labels >= 0) & (labels < NUM_CLASSES)
    x = pred_cls_ref[...]
    col = lax.broadcasted_iota(jnp.int32, x.shape, 1)
    m = (col == labels) & posb
    mf = m.astype(jnp.float32)
    e = jnp.exp(-jnp.abs(x))
    d = 1.0 + e
    r = 1.0 / d
    er = e * r
    ell = jnp.log(d)
    xneg = x < 0.0
    sg = jnp.where(m ^ xneg, er, r)
    sp = jnp.maximum(x, 0.0) - x * mf + ell
    g = sg * sg * sp
    cls_sum = jnp.sum(g)
    clsm_sum = jnp.sum(jnp.where(m, g, 0.0))

    R = jnp.transpose(r_ref[...])
    P = jnp.transpose(p_ref[...])
    G = jnp.transpose(g_ref[...])
    A = jnp.transpose(a_ref[...])
    prx, pry, prw, prh = (R[0:1], R[1:2], R[2:3], R[3:4])
    px1, py1, px2, py2 = (P[0:1], P[1:2], P[2:3], P[3:4])
    gx1, gy1, gx2, gy2 = (G[0:1], G[1:2], G[2:3], G[3:4])
    ax, ay, aw, ah = (A[0:1], A[1:2], A[2:3], A[3:4])

    iw = jnp.clip(jnp.minimum(px2, gx2) - jnp.maximum(px1, gx1), 0.0)
    ih = jnp.clip(jnp.minimum(py2, gy2) - jnp.maximum(py1, gy1), 0.0)
    inter = iw * ih
    a1 = jnp.clip(px2 - px1, 0.0) * jnp.clip(py2 - py1, 0.0)
    a2 = jnp.clip(gx2 - gx1, 0.0) * jnp.clip(gy2 - gy1, 0.0)
    union = a1 + a2 - inter
    iou = inter / jnp.clip(union, 1e-7)
    cw = jnp.maximum(px2, gx2) - jnp.minimum(px1, gx1)
    ch = jnp.maximum(py2, gy2) - jnp.minimum(py1, gy1)
    area_c = jnp.clip(cw, 0.0) * jnp.clip(ch, 0.0)
    giou = iou - (area_c - union) / jnp.clip(area_c, 1e-7)
    reg_sum = jnp.sum((1.0 - giou) * pos_f)

    gw = jnp.clip(gx2 - gx1, 1e-7)
    gh = jnp.clip(gy2 - gy1, 1e-7)
    ecx = ((gx1 + gx2) * 0.5 - ax) / aw
    ecy = ((gy1 + gy2) * 0.5 - ay) / ah
    ew = jnp.log(gw / aw)
    eh = jnp.log(gh / ah)
    l1 = (jnp.abs(prx - ecx) + jnp.abs(pry - ecy)
          + jnp.abs(prw - ew) + jnp.abs(prh - eh))
    box_sum = jnp.sum(l1 * pos_f)

    cls_ref[...] += cls_sum
    clsm_ref[...] += clsm_sum
    reg_ref[...] += reg_sum
    box_ref[...] += box_sum
    npos_ref[...] += jnp.sum(pos_f)


@jax.jit
def kernel(pred_cls, pred_reg, pred_box, gt_box, anchors, tgt_labels):
    lab_f = tgt_labels.astype(jnp.float32).reshape(1, N)
    grid = (N // BLOCK,)
    scalar_spec = pl.BlockSpec((1, 1), lambda i: (0, 0))
    box_spec = pl.BlockSpec((BLOCK, 4), lambda i: (i, 0))
    out = pl.pallas_call(
        _criterion_block,
        grid=grid,
        in_specs=[
            pl.BlockSpec((BLOCK, NUM_CLASSES), lambda i: (i, 0)),
            box_spec, box_spec, box_spec, box_spec,
            pl.BlockSpec((1, BLOCK), lambda i: (0, i)),
        ],
        out_specs=(scalar_spec,) * 5,
        out_shape=tuple(jax.ShapeDtypeStruct((1, 1), jnp.float32)
                        for _ in range(5)),
    )(pred_cls, pred_reg, pred_box, gt_box, anchors, lab_f)
    cls_sum, clsm_sum, reg_sum, box_sum, npos = (o[0, 0] for o in out)
    num_fgs = jnp.maximum(npos, 1.0)
    loss_cls = 0.75 * cls_sum - 0.5 * clsm_sum
    return jnp.stack([loss_cls, reg_sum, box_sum]) / num_fgs
